# baseline (speedup 1.0000x reference)
import jax
import jax.numpy as jnp
from jax import lax
from jax.experimental import pallas as pl
from jax.experimental.pallas import tpu as pltpu
from jax.experimental.pallas import tpu_sc as plsc

B, T = 4096, 50
D = 128
TPAD = 56
NC, NS = 2, 16
NW = NC * NS
BW = B // NW
NBUF = 7
NGROUP = 7


def _emb_body(idx_hbm, table_hbm, out_hbm, idx_v, bufs, sem_g, sem_s):
    wid = lax.axis_index("s") * NC + lax.axis_index("c")
    b0 = wid * BW

    pltpu.sync_copy(idx_hbm.at[:, pl.ds(b0, BW)], idx_v)

    def gather(t, j):
        pltpu.async_copy(table_hbm.at[idx_v.at[t]], bufs.at[j], sem_g.at[j])

    def scatter(t, j):
        pltpu.async_copy(bufs.at[j], out_hbm.at[t, pl.ds(b0, BW)], sem_s.at[j])

    def wait_g(j):
        pltpu.make_async_copy(out_hbm.at[0, pl.ds(0, BW)], bufs.at[j],
                              sem_g.at[j]).wait()

    def wait_s(j):
        pltpu.make_async_copy(bufs.at[j], out_hbm.at[0, pl.ds(0, BW)],
                              sem_s.at[j]).wait()

    for j in range(NBUF):
        gather(j, j)

    def body(g, carry):
        t = g * NBUF
        for j in range(NBUF):
            wait_g(j)
            scatter(t + j, j)
        for j in range(NBUF):
            wait_s(j)
            gather(t + NBUF + j, j)
        return carry

    lax.fori_loop(0, NGROUP - 1, body, 0)

    t = NBUF * (NGROUP - 1)
    for j in range(NBUF):
        wait_g(j)
        scatter(t + j, j)
    wait_s(0)
    gather(T - 1, 0)
    for j in range(1, NBUF):
        wait_s(j)
    wait_g(0)
    scatter(T - 1, 0)
    wait_s(0)


def _embedding_lookup(idx_t, weights):
    mesh = plsc.VectorSubcoreMesh(core_axis_name="c", subcore_axis_name="s")
    k = pl.kernel(
        _emb_body,
        mesh=mesh,
        out_type=jax.ShapeDtypeStruct((T, B, D), jnp.float32),
        scratch_types=[
            pltpu.VMEM((T, BW), jnp.int32),
            pltpu.VMEM((NBUF, BW, D), jnp.float32),
            pltpu.SemaphoreType.DMA((NBUF,)),
            pltpu.SemaphoreType.DMA((NBUF,)),
        ],
        compiler_params=pltpu.CompilerParams(use_tc_tiling_on_sc=True),
    )
    return k(idx_t, weights)


def kernel(token_ids, weights):
    ids_t = token_ids.astype(jnp.int32).T
    out_t = _embedding_lookup(ids_t, weights)
    return jnp.transpose(out_t, (1, 0, 2))


# speedup vs baseline: 1.0134x; 1.0134x over previous
import jax
import jax.numpy as jnp
from jax import lax
from jax.experimental import pallas as pl
from jax.experimental.pallas import tpu as pltpu
from jax.experimental.pallas import tpu_sc as plsc

B, T = 4096, 50
D = 128
TPAD = 56
NC, NS = 2, 16
NW = NC * NS
BW = B // NW
NBUF = 5
NGROUP = T // NBUF


def _emb_body(idx_hbm, table_hbm, out_hbm, idx_v, bufs, sem_g, sem_s):
    wid = lax.axis_index("s") * NC + lax.axis_index("c")
    b0 = wid * BW

    pltpu.sync_copy(idx_hbm.at[:, pl.ds(b0, BW)], idx_v)

    def gather(t, j):
        pltpu.async_copy(table_hbm.at[idx_v.at[t]], bufs.at[j], sem_g.at[j])

    def scatter(t, j):
        pltpu.async_copy(bufs.at[j], out_hbm.at[t, pl.ds(b0, BW)], sem_s.at[j])

    def wait_g(j):
        pltpu.make_async_copy(out_hbm.at[0, pl.ds(0, BW)], bufs.at[j],
                              sem_g.at[j]).wait()

    def wait_s(j):
        pltpu.make_async_copy(bufs.at[j], out_hbm.at[0, pl.ds(0, BW)],
                              sem_s.at[j]).wait()

    for j in range(NBUF):
        gather(j, j)

    def body(g, carry):
        t = g * NBUF
        for j in range(NBUF):
            wait_g(j)
            scatter(t + j, j)
        for j in range(NBUF):
            wait_s(j)
            gather(t + NBUF + j, j)
        return carry

    lax.fori_loop(0, NGROUP - 1, body, 0)

    t = T - NBUF
    for j in range(NBUF):
        wait_g(j)
        scatter(t + j, j)
    for j in range(NBUF):
        wait_s(j)


def _embedding_lookup(idx_t, weights):
    mesh = plsc.VectorSubcoreMesh(core_axis_name="c", subcore_axis_name="s")
    k = pl.kernel(
        _emb_body,
        mesh=mesh,
        out_type=jax.ShapeDtypeStruct((T, B, D), jnp.float32),
        scratch_types=[
            pltpu.VMEM((T, BW), jnp.int32),
            pltpu.VMEM((NBUF, BW, D), jnp.float32),
            pltpu.SemaphoreType.DMA((NBUF,)),
            pltpu.SemaphoreType.DMA((NBUF,)),
        ],
        compiler_params=pltpu.CompilerParams(use_tc_tiling_on_sc=True),
    )
    return k(idx_t, weights)


def kernel(token_ids, weights):
    ids_t = token_ids.astype(jnp.int32).T
    out_t = _embedding_lookup(ids_t, weights)
    return jnp.transpose(out_t, (1, 0, 2))
